# Initial kernel scaffold; baseline (speedup 1.0000x reference)
import jax
import jax.numpy as jnp
from jax.experimental import pallas as pl


def kernel(x, edge_index, W1, b1, W2, b2, Wxz, bxz, Whz, bhz, Wxr, bxr, Whr, bhr, Wxh, bxh, Whh, bhh, Wo, bo):
    raise NotImplementedError("write your pallas kernel here")


# speedup vs baseline: 8.6287x; 8.6287x over previous
import functools

import jax
import jax.numpy as jnp
from jax import lax
from jax.experimental import pallas as pl
from jax.experimental.pallas import tpu as pltpu
from jax.experimental.pallas import tpu_sc as plsc

N = 10000
DIN = 128
C = 128
TILES = 32
RPS = 640
NP = RPS * 16
DEGW = 16
R = 1000
GRID = N // R

_mesh = plsc.VectorSubcoreMesh(core_axis_name="c", subcore_axis_name="s")


def _hist_body(dst2d, ones_hbm, zfeat, out, didx, ones_v, shdeg, sem):
    cpt = dst2d.shape[0] // TILES
    cid = lax.axis_index("c")
    sid = lax.axis_index("s")
    wid = cid * 16 + sid
    rbase = sid * RPS
    pltpu.sync_copy(zfeat.at[pl.ds(rbase, RPS)], shdeg.at[pl.ds(rbase, RPS)])
    pltpu.sync_copy(ones_hbm, ones_v)
    pltpu.sync_copy(dst2d.at[pl.ds(wid * cpt, cpt)], didx)
    plsc.subcore_barrier()

    def body(j, carry):
        pltpu.sync_copy(ones_v, shdeg.at[didx.at[j]], add=True)
        return carry

    lax.fori_loop(0, cpt, body, 0)
    plsc.subcore_barrier()
    pltpu.sync_copy(shdeg.at[pl.ds(rbase, RPS)], out.at[cid, pl.ds(rbase, RPS)])


def _prop_body(xs, src2d, dst2d, zfeat, out, sidx, didx, rows, shacc, sem):
    cpt = src2d.shape[0] // TILES
    cid = lax.axis_index("c")
    sid = lax.axis_index("s")
    wid = cid * 16 + sid
    rbase = sid * RPS
    pltpu.sync_copy(zfeat.at[pl.ds(rbase, RPS)], shacc.at[pl.ds(rbase, RPS)])
    pltpu.sync_copy(src2d.at[pl.ds(wid * cpt, cpt)], sidx)
    pltpu.sync_copy(dst2d.at[pl.ds(wid * cpt, cpt)], didx)
    plsc.subcore_barrier()

    def body(j, carry):
        pltpu.async_copy(xs.at[sidx.at[j]], rows, sem).wait()
        pltpu.sync_copy(rows, shacc.at[didx.at[j]], add=True)
        return carry

    lax.fori_loop(0, cpt, body, 0)
    plsc.subcore_barrier()
    pltpu.sync_copy(shacc.at[pl.ds(rbase, RPS)], out.at[cid, pl.ds(rbase, RPS)])


def _sc_hist(dst2d, ones_hbm, zfeat):
    cpt = dst2d.shape[0] // TILES
    fn = functools.partial(
        pl.kernel,
        mesh=_mesh,
        out_type=jax.ShapeDtypeStruct((2, NP, DIN), jnp.float32),
        scratch_types=[
            pltpu.VMEM((cpt, C), jnp.int32),
            pltpu.VMEM((C, DIN), jnp.float32),
            pltpu.VMEM_SHARED((NP, DIN), jnp.float32),
            pltpu.SemaphoreType.DMA,
        ],
    )(_hist_body)
    return fn(dst2d, ones_hbm, zfeat)


def _sc_prop(xs, src2d, dst2d, zfeat):
    cpt = src2d.shape[0] // TILES
    fn = functools.partial(
        pl.kernel,
        mesh=_mesh,
        out_type=jax.ShapeDtypeStruct((2, NP, DIN), jnp.float32),
        scratch_types=[
            pltpu.VMEM((cpt, C), jnp.int32),
            pltpu.VMEM((cpt, C), jnp.int32),
            pltpu.VMEM((C, DIN), jnp.float32),
            pltpu.VMEM_SHARED((NP, DIN), jnp.float32),
            pltpu.SemaphoreType.DMA,
        ],
    )(_prop_body)
    return fn(xs, src2d, dst2d, zfeat)


def _prep_body(deg_ref, x_ref, xs_ref, dinv_ref):
    deg = deg_ref[0, :, 0:1] + deg_ref[1, :, 0:1] + 1.0
    dinv = lax.rsqrt(deg)
    dinv_ref[...] = jnp.broadcast_to(dinv, dinv_ref.shape)
    xs_ref[...] = x_ref[...] * dinv


def _tc_prep(deg, x):
    return pl.pallas_call(
        _prep_body,
        grid=(GRID,),
        in_specs=[
            pl.BlockSpec((2, R, DIN), lambda r: (0, r, 0)),
            pl.BlockSpec((R, DIN), lambda r: (r, 0)),
        ],
        out_specs=[
            pl.BlockSpec((R, DIN), lambda r: (r, 0)),
            pl.BlockSpec((R, DEGW), lambda r: (r, 0)),
        ],
        out_shape=[
            jax.ShapeDtypeStruct((NP, DIN), jnp.float32),
            jax.ShapeDtypeStruct((N, DEGW), jnp.float32),
        ],
    )(deg, x)


def _mid_body(acc_ref, xs1_ref, dinv_ref, w1_ref, b1_ref, w2_ref, xs2_ref):
    dinv = dinv_ref[:, 0:1]
    s1 = (acc_ref[0] + acc_ref[1] + xs1_ref[...]) * dinv
    f = jnp.maximum(
        jnp.dot(s1, w1_ref[...], preferred_element_type=jnp.float32) + b1_ref[...],
        0.0)
    xw2 = jnp.dot(f, w2_ref[...], preferred_element_type=jnp.float32)
    xs2_ref[...] = xw2 * dinv


def _tc_mid(acc, xs1, dinv, W1, b1, W2):
    return pl.pallas_call(
        _mid_body,
        grid=(GRID,),
        in_specs=[
            pl.BlockSpec((2, R, DIN), lambda r: (0, r, 0)),
            pl.BlockSpec((R, DIN), lambda r: (r, 0)),
            pl.BlockSpec((R, DEGW), lambda r: (r, 0)),
            pl.BlockSpec(W1.shape, lambda r: (0, 0)),
            pl.BlockSpec((1, 256), lambda r: (0, 0)),
            pl.BlockSpec(W2.shape, lambda r: (0, 0)),
        ],
        out_specs=pl.BlockSpec((R, DIN), lambda r: (r, 0)),
        out_shape=jax.ShapeDtypeStruct((NP, DIN), jnp.float32),
    )(acc, xs1, dinv, W1, b1, W2)


def _fin_body(acc_ref, xs2_ref, dinv_ref, x_ref, b2_ref,
              wzz_ref, wzx_ref, bz_ref, whz_ref, whx_ref, bh_ref,
              wo_ref, bo_ref, h_ref, out_ref, zsum_ref):
    r = pl.program_id(0)
    dinv = dinv_ref[:, 0:1]
    z = jnp.maximum(
        (acc_ref[0] + acc_ref[1] + xs2_ref[...]) * dinv + b2_ref[...], 0.0)
    g = jax.nn.sigmoid(
        jnp.dot(z, wzz_ref[...], preferred_element_type=jnp.float32)
        + jnp.dot(x_ref[...], wzx_ref[...], preferred_element_type=jnp.float32)
        + bz_ref[...])
    ht = jnp.tanh(
        jnp.dot(z, whz_ref[...], preferred_element_type=jnp.float32)
        + jnp.dot(x_ref[...], whx_ref[...], preferred_element_type=jnp.float32)
        + bh_ref[...])
    h_ref[...] = (1.0 - g) * ht

    @pl.when(r == 0)
    def _init():
        zsum_ref[...] = jnp.zeros_like(zsum_ref)

    zsum_ref[...] += jnp.sum(z, axis=0, keepdims=True)

    @pl.when(r == GRID - 1)
    def _head():
        out_ref[...] = (
            jnp.dot(zsum_ref[...] * (1.0 / N), wo_ref[...],
                    preferred_element_type=jnp.float32) + bo_ref[...])


def _tc_fin(acc, xs2, dinv, x, b2, Wzz, Wzx, bz, Whz_, Whx, bh, Wo, bo):
    return pl.pallas_call(
        _fin_body,
        grid=(GRID,),
        in_specs=[
            pl.BlockSpec((2, R, DIN), lambda r: (0, r, 0)),
            pl.BlockSpec((R, DIN), lambda r: (r, 0)),
            pl.BlockSpec((R, DEGW), lambda r: (r, 0)),
            pl.BlockSpec((R, DIN), lambda r: (r, 0)),
            pl.BlockSpec((1, DIN), lambda r: (0, 0)),
            pl.BlockSpec(Wzz.shape, lambda r: (0, 0)),
            pl.BlockSpec(Wzx.shape, lambda r: (0, 0)),
            pl.BlockSpec((1, 256), lambda r: (0, 0)),
            pl.BlockSpec(Whz_.shape, lambda r: (0, 0)),
            pl.BlockSpec(Whx.shape, lambda r: (0, 0)),
            pl.BlockSpec((1, 256), lambda r: (0, 0)),
            pl.BlockSpec(Wo.shape, lambda r: (0, 0)),
            pl.BlockSpec((1, 1), lambda r: (0, 0)),
        ],
        out_specs=[
            pl.BlockSpec((R, 256), lambda r: (r, 0)),
            pl.BlockSpec((1, 1), lambda r: (0, 0)),
            pl.BlockSpec((1, DIN), lambda r: (0, 0)),
        ],
        out_shape=[
            jax.ShapeDtypeStruct((N, 256), jnp.float32),
            jax.ShapeDtypeStruct((1, 1), jnp.float32),
            jax.ShapeDtypeStruct((1, DIN), jnp.float32),
        ],
    )(acc, xs2, dinv, x, b2, Wzz, Wzx, bz, Whz_, Whx, bh, Wo, bo)


def kernel(x, edge_index, W1, b1, W2, b2, Wxz, bxz, Whz, bhz, Wxr, bxr,
           Whr, bhr, Wxh, bxh, Whh, bhh, Wo, bo):
    e = edge_index.shape[1]
    cpt = -(-e // (TILES * C))
    cpt = (cpt + 7) // 8 * 8
    ep = TILES * cpt * C
    src = edge_index[0].astype(jnp.int32)
    dst = edge_index[1].astype(jnp.int32)
    fill = jnp.full((ep - e,), N, jnp.int32)
    src2d = jnp.concatenate([src, fill]).reshape(ep // C, C)
    dst2d = jnp.concatenate([dst, fill]).reshape(ep // C, C)

    ones_hbm = jnp.ones((C, DIN), jnp.float32)
    zfeat = jnp.zeros((NP, DIN), jnp.float32)

    deg = _sc_hist(dst2d, ones_hbm, zfeat)
    xs1, dinv = _tc_prep(deg, x)
    acc1 = _sc_prop(xs1, src2d, dst2d, zfeat)
    xs2 = _tc_mid(acc1, xs1[:N], dinv, W1, b1.reshape(1, 256), W2)
    acc2 = _sc_prop(xs2, src2d, dst2d, zfeat)
    H, out, _ = _tc_fin(
        acc2, xs2[:N], dinv, x, b2.reshape(1, DIN),
        Wxz[:DIN], Wxz[DIN:], (bxz + bhz).reshape(1, 256),
        Wxh[:DIN], Wxh[DIN:], (bxh + bhh).reshape(1, 256),
        Wo, bo.reshape(1, 1))
    return (out.reshape(1), H)


# speedup vs baseline: 20.2893x; 2.3514x over previous
import functools

import jax
import jax.numpy as jnp
from jax import lax
from jax.experimental import pallas as pl
from jax.experimental.pallas import tpu as pltpu
from jax.experimental.pallas import tpu_sc as plsc

N = 10000
DIN = 128
C = 128
TILES = 32
RPS = 640
NP = RPS * 16
DEGW = 16
R = 1000
GRID = N // R

_mesh = plsc.VectorSubcoreMesh(core_axis_name="c", subcore_axis_name="s")


def _hist_body(dst2d, ones_hbm, zfeat, out, didx, ones_v, shdeg, sem):
    cpt = dst2d.shape[0] // TILES
    cid = lax.axis_index("c")
    sid = lax.axis_index("s")
    wid = cid * 16 + sid
    rbase = sid * RPS
    pltpu.sync_copy(zfeat.at[pl.ds(rbase, RPS)], shdeg.at[pl.ds(rbase, RPS)])
    pltpu.sync_copy(ones_hbm, ones_v)
    pltpu.sync_copy(dst2d.at[pl.ds(wid * cpt, cpt)], didx)
    plsc.subcore_barrier()

    def body(j, carry):
        pltpu.sync_copy(ones_v, shdeg.at[didx.at[j]], add=True)
        return carry

    lax.fori_loop(0, cpt, body, 0)
    plsc.subcore_barrier()
    pltpu.sync_copy(shdeg.at[pl.ds(rbase, RPS)], out.at[cid, pl.ds(rbase, RPS)])


def _prop_body(xs, src2d, dst2d, zfeat, out, sidx, didx, rows, shacc, sem):
    cpt = src2d.shape[0] // TILES
    cid = lax.axis_index("c")
    sid = lax.axis_index("s")
    wid = cid * 16 + sid
    rbase = sid * RPS
    pltpu.sync_copy(zfeat.at[pl.ds(rbase, RPS)], shacc.at[pl.ds(rbase, RPS)])
    pltpu.sync_copy(src2d.at[pl.ds(wid * cpt, cpt)], sidx)
    pltpu.sync_copy(dst2d.at[pl.ds(wid * cpt, cpt)], didx)
    plsc.subcore_barrier()

    def body(j, carry):
        pltpu.async_copy(xs.at[sidx.at[j]], rows, sem).wait()
        pltpu.sync_copy(rows, shacc.at[didx.at[j]], add=True)
        return carry

    lax.fori_loop(0, cpt, body, 0)
    plsc.subcore_barrier()
    pltpu.sync_copy(shacc.at[pl.ds(rbase, RPS)], out.at[cid, pl.ds(rbase, RPS)])


def _sc_hist(dst2d, ones_hbm, zfeat):
    cpt = dst2d.shape[0] // TILES
    fn = functools.partial(
        pl.kernel,
        mesh=_mesh,
        out_type=jax.ShapeDtypeStruct((2, NP, DIN), jnp.float32),
        scratch_types=[
            pltpu.VMEM((cpt, C), jnp.int32),
            pltpu.VMEM((C, DIN), jnp.float32),
            pltpu.VMEM_SHARED((NP, DIN), jnp.float32),
            pltpu.SemaphoreType.DMA,
        ],
    )(_hist_body)
    return fn(dst2d, ones_hbm, zfeat)


def _sc_prop(xs, src2d, dst2d, zfeat):
    cpt = src2d.shape[0] // TILES
    fn = functools.partial(
        pl.kernel,
        mesh=_mesh,
        out_type=jax.ShapeDtypeStruct((2, NP, DIN), jnp.float32),
        scratch_types=[
            pltpu.VMEM((cpt, C), jnp.int32),
            pltpu.VMEM((cpt, C), jnp.int32),
            pltpu.VMEM((C, DIN), jnp.float32),
            pltpu.VMEM_SHARED((NP, DIN), jnp.float32),
            pltpu.SemaphoreType.DMA,
        ],
    )(_prop_body)
    return fn(xs, src2d, dst2d, zfeat)


def _prep_body(deg_ref, x_ref, xs_ref, dinv_ref):
    deg = deg_ref[0, :, 0:1] + deg_ref[1, :, 0:1] + 1.0
    dinv = lax.rsqrt(deg)
    dinv_ref[...] = jnp.broadcast_to(dinv, dinv_ref.shape)
    xs_ref[...] = x_ref[...] * dinv


def _tc_prep(deg, x):
    return pl.pallas_call(
        _prep_body,
        grid=(GRID,),
        in_specs=[
            pl.BlockSpec((2, R, DIN), lambda r: (0, r, 0)),
            pl.BlockSpec((R, DIN), lambda r: (r, 0)),
        ],
        out_specs=[
            pl.BlockSpec((R, DIN), lambda r: (r, 0)),
            pl.BlockSpec((R, DEGW), lambda r: (r, 0)),
        ],
        out_shape=[
            jax.ShapeDtypeStruct((NP, DIN), jnp.float32),
            jax.ShapeDtypeStruct((N, DEGW), jnp.float32),
        ],
    )(deg, x)


def _mid_body(acc_ref, xs1_ref, dinv_ref, w1_ref, b1_ref, w2_ref, xs2_ref):
    dinv = dinv_ref[:, 0:1]
    s1 = (acc_ref[0] + acc_ref[1] + xs1_ref[...]) * dinv
    f = jnp.maximum(
        jnp.dot(s1, w1_ref[...], preferred_element_type=jnp.float32) + b1_ref[...],
        0.0)
    xw2 = jnp.dot(f, w2_ref[...], preferred_element_type=jnp.float32)
    xs2_ref[...] = xw2 * dinv


def _tc_mid(acc, xs1, dinv, W1, b1, W2):
    return pl.pallas_call(
        _mid_body,
        grid=(GRID,),
        in_specs=[
            pl.BlockSpec((2, R, DIN), lambda r: (0, r, 0)),
            pl.BlockSpec((R, DIN), lambda r: (r, 0)),
            pl.BlockSpec((R, DEGW), lambda r: (r, 0)),
            pl.BlockSpec(W1.shape, lambda r: (0, 0)),
            pl.BlockSpec((1, 256), lambda r: (0, 0)),
            pl.BlockSpec(W2.shape, lambda r: (0, 0)),
        ],
        out_specs=pl.BlockSpec((R, DIN), lambda r: (r, 0)),
        out_shape=jax.ShapeDtypeStruct((NP, DIN), jnp.float32),
    )(acc, xs1, dinv, W1, b1, W2)


def _fin_body(acc_ref, xs2_ref, dinv_ref, x_ref, b2_ref,
              wzz_ref, wzx_ref, bz_ref, whz_ref, whx_ref, bh_ref,
              wo_ref, bo_ref, h_ref, out_ref, zsum_ref):
    r = pl.program_id(0)
    dinv = dinv_ref[:, 0:1]
    z = jnp.maximum(
        (acc_ref[0] + acc_ref[1] + xs2_ref[...]) * dinv + b2_ref[...], 0.0)
    g = jax.nn.sigmoid(
        jnp.dot(z, wzz_ref[...], preferred_element_type=jnp.float32)
        + jnp.dot(x_ref[...], wzx_ref[...], preferred_element_type=jnp.float32)
        + bz_ref[...])
    ht = jnp.tanh(
        jnp.dot(z, whz_ref[...], preferred_element_type=jnp.float32)
        + jnp.dot(x_ref[...], whx_ref[...], preferred_element_type=jnp.float32)
        + bh_ref[...])
    h_ref[...] = (1.0 - g) * ht

    @pl.when(r == 0)
    def _init():
        zsum_ref[...] = jnp.zeros_like(zsum_ref)

    zsum_ref[...] += jnp.sum(z, axis=0, keepdims=True)

    @pl.when(r == GRID - 1)
    def _head():
        out_ref[...] = (
            jnp.dot(zsum_ref[...] * (1.0 / N), wo_ref[...],
                    preferred_element_type=jnp.float32) + bo_ref[...])


def _tc_fin(acc, xs2, dinv, x, b2, Wzz, Wzx, bz, Whz_, Whx, bh, Wo, bo):
    return pl.pallas_call(
        _fin_body,
        grid=(GRID,),
        in_specs=[
            pl.BlockSpec((2, R, DIN), lambda r: (0, r, 0)),
            pl.BlockSpec((R, DIN), lambda r: (r, 0)),
            pl.BlockSpec((R, DEGW), lambda r: (r, 0)),
            pl.BlockSpec((R, DIN), lambda r: (r, 0)),
            pl.BlockSpec((1, DIN), lambda r: (0, 0)),
            pl.BlockSpec(Wzz.shape, lambda r: (0, 0)),
            pl.BlockSpec(Wzx.shape, lambda r: (0, 0)),
            pl.BlockSpec((1, 256), lambda r: (0, 0)),
            pl.BlockSpec(Whz_.shape, lambda r: (0, 0)),
            pl.BlockSpec(Whx.shape, lambda r: (0, 0)),
            pl.BlockSpec((1, 256), lambda r: (0, 0)),
            pl.BlockSpec(Wo.shape, lambda r: (0, 0)),
            pl.BlockSpec((1, 1), lambda r: (0, 0)),
        ],
        out_specs=[
            pl.BlockSpec((R, 256), lambda r: (r, 0)),
            pl.BlockSpec((1, 1), lambda r: (0, 0)),
            pl.BlockSpec((1, DIN), lambda r: (0, 0)),
        ],
        out_shape=[
            jax.ShapeDtypeStruct((N, 256), jnp.float32),
            jax.ShapeDtypeStruct((1, 1), jnp.float32),
            jax.ShapeDtypeStruct((1, DIN), jnp.float32),
        ],
    )(acc, xs2, dinv, x, b2, Wzz, Wzx, bz, Whz_, Whx, bh, Wo, bo)


def kernel(x, edge_index, W1, b1, W2, b2, Wxz, bxz, Whz, bhz, Wxr, bxr,
           Whr, bhr, Wxh, bxh, Whh, bhh, Wo, bo):
    e = edge_index.shape[1]
    cpt = -(-e // (TILES * C))
    cpt = (cpt + 7) // 8 * 8
    ep = TILES * cpt * C
    src = edge_index[0].astype(jnp.int32)
    dst = edge_index[1].astype(jnp.int32)
    pad_i = jnp.arange(ep - e, dtype=jnp.int32)
    src2d = jnp.concatenate([src, pad_i % 256]).reshape(ep // C, C)
    dst2d = jnp.concatenate([dst, N + pad_i % (NP - N)]).reshape(ep // C, C)

    ones_hbm = jnp.ones((C, DIN), jnp.float32)
    zfeat = jnp.zeros((NP, DIN), jnp.float32)

    deg = _sc_hist(dst2d, ones_hbm, zfeat)
    xs1, dinv = _tc_prep(deg, x)
    acc1 = _sc_prop(xs1, src2d, dst2d, zfeat)
    xs2 = _tc_mid(acc1, xs1[:N], dinv, W1, b1.reshape(1, 256), W2)
    acc2 = _sc_prop(xs2, src2d, dst2d, zfeat)
    H, out, _ = _tc_fin(
        acc2, xs2[:N], dinv, x, b2.reshape(1, DIN),
        Wxz[:DIN], Wxz[DIN:], (bxz + bhz).reshape(1, 256),
        Wxh[:DIN], Wxh[DIN:], (bxh + bhh).reshape(1, 256),
        Wo, bo.reshape(1, 1))
    return (out.reshape(1), H)


# speedup vs baseline: 24.6161x; 1.2133x over previous
import functools

import jax
import jax.numpy as jnp
from jax import lax
from jax.experimental import pallas as pl
from jax.experimental.pallas import tpu as pltpu
from jax.experimental.pallas import tpu_sc as plsc

N = 10000
DIN = 128
C = 128
TILES = 32
RPS = 640
NP = RPS * 16
DEGW = 16
R = 1000
GRID = N // R

_mesh = plsc.VectorSubcoreMesh(core_axis_name="c", subcore_axis_name="s")


def _hist_body(dst2d, ones_hbm, zfeat, out, didx, ones_v, shdeg, sem):
    cpt = dst2d.shape[0] // TILES
    cid = lax.axis_index("c")
    sid = lax.axis_index("s")
    wid = cid * 16 + sid
    rbase = sid * RPS
    pltpu.sync_copy(zfeat.at[pl.ds(rbase, RPS)], shdeg.at[pl.ds(rbase, RPS)])
    pltpu.sync_copy(ones_hbm, ones_v)
    pltpu.sync_copy(dst2d.at[pl.ds(wid * cpt, cpt)], didx)
    plsc.subcore_barrier()

    def body(j, carry):
        pltpu.sync_copy(ones_v, shdeg.at[didx.at[j]], add=True)
        return carry

    lax.fori_loop(0, cpt, body, 0)
    plsc.subcore_barrier()
    pltpu.sync_copy(shdeg.at[pl.ds(rbase, RPS)], out.at[cid, pl.ds(rbase, RPS)])


def _prop_body(xs, src2d, dst2d, zfeat, out, sidx, didx,
               rows0, rows1, shacc, sg0, sg1, ss0, ss1):
    cpt = src2d.shape[0] // TILES
    half = cpt // 2
    npair = half // 2
    cid = lax.axis_index("c")
    sid = lax.axis_index("s")
    wid = cid * 16 + sid
    rbase = sid * RPS
    pltpu.sync_copy(zfeat.at[pl.ds(rbase, RPS)], shacc.at[pl.ds(rbase, RPS)])
    plsc.subcore_barrier()

    for h in range(2):
        pltpu.sync_copy(src2d.at[pl.ds((wid * 2 + h) * half, half)], sidx)
        pltpu.sync_copy(dst2d.at[pl.ds((wid * 2 + h) * half, half)], didx)
        pltpu.async_copy(xs.at[sidx.at[0]], rows0, sg0)

        def pair(k, carry):
            j0 = k * 2
            j1 = j0 + 1
            pltpu.make_async_copy(xs.at[sidx.at[j0]], rows0, sg0).wait()
            pltpu.async_copy(rows0, shacc.at[didx.at[j0]], ss0, add=True)

            @pl.when(k > 0)
            def _():
                pltpu.make_async_copy(
                    rows1, shacc.at[didx.at[j0 - 1]], ss1).wait()

            pltpu.async_copy(xs.at[sidx.at[j1]], rows1, sg1)
            pltpu.make_async_copy(xs.at[sidx.at[j1]], rows1, sg1).wait()
            pltpu.async_copy(rows1, shacc.at[didx.at[j1]], ss1, add=True)
            pltpu.make_async_copy(rows0, shacc.at[didx.at[j0]], ss0).wait()

            @pl.when(k < npair - 1)
            def _():
                pltpu.async_copy(xs.at[sidx.at[j0 + 2]], rows0, sg0)

            return carry

        lax.fori_loop(0, npair, pair, 0)
        pltpu.make_async_copy(rows1, shacc.at[didx.at[half - 1]], ss1).wait()

    plsc.subcore_barrier()
    pltpu.sync_copy(shacc.at[pl.ds(rbase, RPS)], out.at[cid, pl.ds(rbase, RPS)])


def _sc_hist(dst2d, ones_hbm, zfeat):
    cpt = dst2d.shape[0] // TILES
    fn = functools.partial(
        pl.kernel,
        mesh=_mesh,
        out_type=jax.ShapeDtypeStruct((2, NP, DIN), jnp.float32),
        scratch_types=[
            pltpu.VMEM((cpt, C), jnp.int32),
            pltpu.VMEM((C, DIN), jnp.float32),
            pltpu.VMEM_SHARED((NP, DIN), jnp.float32),
            pltpu.SemaphoreType.DMA,
        ],
    )(_hist_body)
    return fn(dst2d, ones_hbm, zfeat)


def _sc_prop(xs, src2d, dst2d, zfeat):
    cpt = src2d.shape[0] // TILES
    fn = functools.partial(
        pl.kernel,
        mesh=_mesh,
        out_type=jax.ShapeDtypeStruct((2, NP, DIN), jnp.float32),
        scratch_types=[
            pltpu.VMEM((cpt // 2, C), jnp.int32),
            pltpu.VMEM((cpt // 2, C), jnp.int32),
            pltpu.VMEM((C, DIN), jnp.float32),
            pltpu.VMEM((C, DIN), jnp.float32),
            pltpu.VMEM_SHARED((NP, DIN), jnp.float32),
            pltpu.SemaphoreType.DMA,
            pltpu.SemaphoreType.DMA,
            pltpu.SemaphoreType.DMA,
            pltpu.SemaphoreType.DMA,
        ],
    )(_prop_body)
    return fn(xs, src2d, dst2d, zfeat)


def _prep_body(deg_ref, x_ref, xs_ref, dinv_ref):
    deg = deg_ref[0, :, 0:1] + deg_ref[1, :, 0:1] + 1.0
    dinv = lax.rsqrt(deg)
    dinv_ref[...] = jnp.broadcast_to(dinv, dinv_ref.shape)
    xs_ref[...] = x_ref[...] * dinv


def _tc_prep(deg, x):
    return pl.pallas_call(
        _prep_body,
        grid=(GRID,),
        in_specs=[
            pl.BlockSpec((2, R, DIN), lambda r: (0, r, 0)),
            pl.BlockSpec((R, DIN), lambda r: (r, 0)),
        ],
        out_specs=[
            pl.BlockSpec((R, DIN), lambda r: (r, 0)),
            pl.BlockSpec((R, DEGW), lambda r: (r, 0)),
        ],
        out_shape=[
            jax.ShapeDtypeStruct((NP, DIN), jnp.float32),
            jax.ShapeDtypeStruct((N, DEGW), jnp.float32),
        ],
    )(deg, x)


def _mid_body(acc_ref, xs1_ref, dinv_ref, w1_ref, b1_ref, w2_ref, xs2_ref):
    dinv = dinv_ref[:, 0:1]
    s1 = (acc_ref[0] + acc_ref[1] + xs1_ref[...]) * dinv
    f = jnp.maximum(
        jnp.dot(s1, w1_ref[...], preferred_element_type=jnp.float32) + b1_ref[...],
        0.0)
    xw2 = jnp.dot(f, w2_ref[...], preferred_element_type=jnp.float32)
    xs2_ref[...] = xw2 * dinv


def _tc_mid(acc, xs1, dinv, W1, b1, W2):
    return pl.pallas_call(
        _mid_body,
        grid=(GRID,),
        in_specs=[
            pl.BlockSpec((2, R, DIN), lambda r: (0, r, 0)),
            pl.BlockSpec((R, DIN), lambda r: (r, 0)),
            pl.BlockSpec((R, DEGW), lambda r: (r, 0)),
            pl.BlockSpec(W1.shape, lambda r: (0, 0)),
            pl.BlockSpec((1, 256), lambda r: (0, 0)),
            pl.BlockSpec(W2.shape, lambda r: (0, 0)),
        ],
        out_specs=pl.BlockSpec((R, DIN), lambda r: (r, 0)),
        out_shape=jax.ShapeDtypeStruct((NP, DIN), jnp.float32),
    )(acc, xs1, dinv, W1, b1, W2)


def _fin_body(acc_ref, xs2_ref, dinv_ref, x_ref, b2_ref,
              wzz_ref, wzx_ref, bz_ref, whz_ref, whx_ref, bh_ref,
              wo_ref, bo_ref, h_ref, out_ref, zsum_ref):
    r = pl.program_id(0)
    dinv = dinv_ref[:, 0:1]
    z = jnp.maximum(
        (acc_ref[0] + acc_ref[1] + xs2_ref[...]) * dinv + b2_ref[...], 0.0)
    g = jax.nn.sigmoid(
        jnp.dot(z, wzz_ref[...], preferred_element_type=jnp.float32)
        + jnp.dot(x_ref[...], wzx_ref[...], preferred_element_type=jnp.float32)
        + bz_ref[...])
    ht = jnp.tanh(
        jnp.dot(z, whz_ref[...], preferred_element_type=jnp.float32)
        + jnp.dot(x_ref[...], whx_ref[...], preferred_element_type=jnp.float32)
        + bh_ref[...])
    h_ref[...] = (1.0 - g) * ht

    @pl.when(r == 0)
    def _init():
        zsum_ref[...] = jnp.zeros_like(zsum_ref)

    zsum_ref[...] += jnp.sum(z, axis=0, keepdims=True)

    @pl.when(r == GRID - 1)
    def _head():
        out_ref[...] = (
            jnp.dot(zsum_ref[...] * (1.0 / N), wo_ref[...],
                    preferred_element_type=jnp.float32) + bo_ref[...])


def _tc_fin(acc, xs2, dinv, x, b2, Wzz, Wzx, bz, Whz_, Whx, bh, Wo, bo):
    return pl.pallas_call(
        _fin_body,
        grid=(GRID,),
        in_specs=[
            pl.BlockSpec((2, R, DIN), lambda r: (0, r, 0)),
            pl.BlockSpec((R, DIN), lambda r: (r, 0)),
            pl.BlockSpec((R, DEGW), lambda r: (r, 0)),
            pl.BlockSpec((R, DIN), lambda r: (r, 0)),
            pl.BlockSpec((1, DIN), lambda r: (0, 0)),
            pl.BlockSpec(Wzz.shape, lambda r: (0, 0)),
            pl.BlockSpec(Wzx.shape, lambda r: (0, 0)),
            pl.BlockSpec((1, 256), lambda r: (0, 0)),
            pl.BlockSpec(Whz_.shape, lambda r: (0, 0)),
            pl.BlockSpec(Whx.shape, lambda r: (0, 0)),
            pl.BlockSpec((1, 256), lambda r: (0, 0)),
            pl.BlockSpec(Wo.shape, lambda r: (0, 0)),
            pl.BlockSpec((1, 1), lambda r: (0, 0)),
        ],
        out_specs=[
            pl.BlockSpec((R, 256), lambda r: (r, 0)),
            pl.BlockSpec((1, 1), lambda r: (0, 0)),
            pl.BlockSpec((1, DIN), lambda r: (0, 0)),
        ],
        out_shape=[
            jax.ShapeDtypeStruct((N, 256), jnp.float32),
            jax.ShapeDtypeStruct((1, 1), jnp.float32),
            jax.ShapeDtypeStruct((1, DIN), jnp.float32),
        ],
    )(acc, xs2, dinv, x, b2, Wzz, Wzx, bz, Whz_, Whx, bh, Wo, bo)


def kernel(x, edge_index, W1, b1, W2, b2, Wxz, bxz, Whz, bhz, Wxr, bxr,
           Whr, bhr, Wxh, bxh, Whh, bhh, Wo, bo):
    e = edge_index.shape[1]
    cpt = -(-e // (TILES * C))
    cpt = (cpt + 7) // 8 * 8
    ep = TILES * cpt * C
    src = edge_index[0].astype(jnp.int32)
    dst = edge_index[1].astype(jnp.int32)
    pad_i = jnp.arange(ep - e, dtype=jnp.int32)
    src2d = jnp.concatenate([src, pad_i % 256]).reshape(ep // C, C)
    dst2d = jnp.concatenate([dst, N + pad_i % (NP - N)]).reshape(ep // C, C)

    ones_hbm = jnp.ones((C, DIN), jnp.float32)
    zfeat = jnp.zeros((NP, DIN), jnp.float32)

    deg = _sc_hist(dst2d, ones_hbm, zfeat)
    xs1, dinv = _tc_prep(deg, x)
    acc1 = _sc_prop(xs1, src2d, dst2d, zfeat)
    xs2 = _tc_mid(acc1, xs1[:N], dinv, W1, b1.reshape(1, 256), W2)
    acc2 = _sc_prop(xs2, src2d, dst2d, zfeat)
    H, out, _ = _tc_fin(
        acc2, xs2[:N], dinv, x, b2.reshape(1, DIN),
        Wxz[:DIN], Wxz[DIN:], (bxz + bhz).reshape(1, 256),
        Wxh[:DIN], Wxh[DIN:], (bxh + bhh).reshape(1, 256),
        Wo, bo.reshape(1, 1))
    return (out.reshape(1), H)


# speedup vs baseline: 28.3670x; 1.1524x over previous
import functools

import jax
import jax.numpy as jnp
from jax import lax
from jax.experimental import pallas as pl
from jax.experimental.pallas import tpu as pltpu
from jax.experimental.pallas import tpu_sc as plsc

N = 10000
DIN = 128
C = 128
TILES = 32
RPS = 640
NP = RPS * 16
DEGW = 16
R = 1000
GRID = N // R

_mesh = plsc.VectorSubcoreMesh(core_axis_name="c", subcore_axis_name="s")


def _hist_body(dst2d, zfeat, out, didx, hist2, outrows, idv, shdeg, sem):
    cpt = dst2d.shape[0] // TILES
    nfr = NP // 128
    cid = lax.axis_index("c")
    sid = lax.axis_index("s")
    wid = cid * 16 + sid
    pltpu.sync_copy(zfeat.at[pl.ds(0, nfr)], hist2)

    @pl.when(sid == 0)
    def _zero_shared():
        pltpu.sync_copy(zfeat.at[pl.ds(0, nfr)], shdeg)

    pltpu.sync_copy(dst2d.at[pl.ds(wid * cpt, cpt)], didx)
    iota = lax.iota(jnp.int32, 16)
    for m in range(nfr // 16):
        idv[pl.ds(m * 16, 16)] = iota + m * 16
    ones = jnp.ones((16,), jnp.float32)

    def body(j, carry):
        for k in range(8):
            v = didx[j, pl.ds(k * 16, 16)]
            plsc.addupdate_scatter(hist2, [v >> 7, v & 127], ones)
        return carry

    lax.fori_loop(0, cpt, body, 0)
    plsc.subcore_barrier()
    pltpu.sync_copy(hist2, shdeg.at[idv], add=True)
    plsc.subcore_barrier()

    @pl.when(sid < 10)
    def _expand():
        pltpu.sync_copy(shdeg.at[pl.ds(sid * 8, 8)], hist2.at[pl.ds(0, 8)])
        zero = jnp.zeros((16,), jnp.int32)
        for r in range(8):
            for m in range(8):
                val = hist2[r, pl.ds(m * 16, 16)]
                plsc.store_scatter(outrows, [iota + m * 16, zero], val)
            pltpu.sync_copy(
                outrows, out.at[cid, pl.ds(sid * 1024 + r * 128, 128)])


def _prop_body(xs, src2d, dst2d, zfeat, out, sidx, didx,
               rows0, rows1, shacc, sg0, sg1, ss0, ss1):
    cpt = src2d.shape[0] // TILES
    half = cpt // 2
    npair = half // 2
    cid = lax.axis_index("c")
    sid = lax.axis_index("s")
    wid = cid * 16 + sid
    rbase = sid * RPS
    pltpu.sync_copy(zfeat.at[pl.ds(rbase, RPS)], shacc.at[pl.ds(rbase, RPS)])
    plsc.subcore_barrier()

    for h in range(2):
        pltpu.sync_copy(src2d.at[pl.ds((wid * 2 + h) * half, half)], sidx)
        pltpu.sync_copy(dst2d.at[pl.ds((wid * 2 + h) * half, half)], didx)
        pltpu.async_copy(xs.at[sidx.at[0]], rows0, sg0)

        def pair(k, carry):
            j0 = k * 2
            j1 = j0 + 1
            pltpu.make_async_copy(xs.at[sidx.at[j0]], rows0, sg0).wait()
            pltpu.async_copy(rows0, shacc.at[didx.at[j0]], ss0, add=True)

            @pl.when(k > 0)
            def _():
                pltpu.make_async_copy(
                    rows1, shacc.at[didx.at[j0 - 1]], ss1).wait()

            pltpu.async_copy(xs.at[sidx.at[j1]], rows1, sg1)
            pltpu.make_async_copy(xs.at[sidx.at[j1]], rows1, sg1).wait()
            pltpu.async_copy(rows1, shacc.at[didx.at[j1]], ss1, add=True)
            pltpu.make_async_copy(rows0, shacc.at[didx.at[j0]], ss0).wait()

            @pl.when(k < npair - 1)
            def _():
                pltpu.async_copy(xs.at[sidx.at[j0 + 2]], rows0, sg0)

            return carry

        lax.fori_loop(0, npair, pair, 0)
        pltpu.make_async_copy(rows1, shacc.at[didx.at[half - 1]], ss1).wait()

    plsc.subcore_barrier()
    pltpu.sync_copy(shacc.at[pl.ds(rbase, RPS)], out.at[cid, pl.ds(rbase, RPS)])


def _sc_hist(dst2d, zfeat):
    cpt = dst2d.shape[0] // TILES
    fn = functools.partial(
        pl.kernel,
        mesh=_mesh,
        out_type=jax.ShapeDtypeStruct((2, NP, 8), jnp.float32),
        scratch_types=[
            pltpu.VMEM((cpt, C), jnp.int32),
            pltpu.VMEM((NP // 128, 128), jnp.float32),
            pltpu.VMEM((128, 8), jnp.float32),
            pltpu.VMEM((NP // 128,), jnp.int32),
            pltpu.VMEM_SHARED((NP // 128, 128), jnp.float32),
            pltpu.SemaphoreType.DMA,
        ],
        compiler_params=pltpu.CompilerParams(needs_layout_passes=False),
    )(_hist_body)
    return fn(dst2d, zfeat)


def _sc_prop(xs, src2d, dst2d, zfeat):
    cpt = src2d.shape[0] // TILES
    fn = functools.partial(
        pl.kernel,
        mesh=_mesh,
        out_type=jax.ShapeDtypeStruct((2, NP, DIN), jnp.float32),
        scratch_types=[
            pltpu.VMEM((cpt // 2, C), jnp.int32),
            pltpu.VMEM((cpt // 2, C), jnp.int32),
            pltpu.VMEM((C, DIN), jnp.float32),
            pltpu.VMEM((C, DIN), jnp.float32),
            pltpu.VMEM_SHARED((NP, DIN), jnp.float32),
            pltpu.SemaphoreType.DMA,
            pltpu.SemaphoreType.DMA,
            pltpu.SemaphoreType.DMA,
            pltpu.SemaphoreType.DMA,
        ],
    )(_prop_body)
    return fn(xs, src2d, dst2d, zfeat)


def _prep_body(deg_ref, x_ref, xs_ref, dinv_ref):
    deg = deg_ref[0, :, 0:1] + deg_ref[1, :, 0:1] + 1.0
    dinv = lax.rsqrt(deg)
    dinv_ref[...] = jnp.broadcast_to(dinv, dinv_ref.shape)
    xs_ref[...] = x_ref[...] * dinv


def _tc_prep(deg, x):
    return pl.pallas_call(
        _prep_body,
        grid=(GRID,),
        in_specs=[
            pl.BlockSpec((2, R, 8), lambda r: (0, r, 0)),
            pl.BlockSpec((R, DIN), lambda r: (r, 0)),
        ],
        out_specs=[
            pl.BlockSpec((R, DIN), lambda r: (r, 0)),
            pl.BlockSpec((R, DEGW), lambda r: (r, 0)),
        ],
        out_shape=[
            jax.ShapeDtypeStruct((NP, DIN), jnp.float32),
            jax.ShapeDtypeStruct((N, DEGW), jnp.float32),
        ],
    )(deg, x)


def _mid_body(acc_ref, xs1_ref, dinv_ref, w1_ref, b1_ref, w2_ref, xs2_ref):
    dinv = dinv_ref[:, 0:1]
    s1 = (acc_ref[0] + acc_ref[1] + xs1_ref[...]) * dinv
    f = jnp.maximum(
        jnp.dot(s1, w1_ref[...], preferred_element_type=jnp.float32) + b1_ref[...],
        0.0)
    xw2 = jnp.dot(f, w2_ref[...], preferred_element_type=jnp.float32)
    xs2_ref[...] = xw2 * dinv


def _tc_mid(acc, xs1, dinv, W1, b1, W2):
    return pl.pallas_call(
        _mid_body,
        grid=(GRID,),
        in_specs=[
            pl.BlockSpec((2, R, DIN), lambda r: (0, r, 0)),
            pl.BlockSpec((R, DIN), lambda r: (r, 0)),
            pl.BlockSpec((R, DEGW), lambda r: (r, 0)),
            pl.BlockSpec(W1.shape, lambda r: (0, 0)),
            pl.BlockSpec((1, 256), lambda r: (0, 0)),
            pl.BlockSpec(W2.shape, lambda r: (0, 0)),
        ],
        out_specs=pl.BlockSpec((R, DIN), lambda r: (r, 0)),
        out_shape=jax.ShapeDtypeStruct((NP, DIN), jnp.float32),
    )(acc, xs1, dinv, W1, b1, W2)


def _fin_body(acc_ref, xs2_ref, dinv_ref, x_ref, b2_ref,
              wzz_ref, wzx_ref, bz_ref, whz_ref, whx_ref, bh_ref,
              wo_ref, bo_ref, h_ref, out_ref, zsum_ref):
    r = pl.program_id(0)
    dinv = dinv_ref[:, 0:1]
    z = jnp.maximum(
        (acc_ref[0] + acc_ref[1] + xs2_ref[...]) * dinv + b2_ref[...], 0.0)
    g = jax.nn.sigmoid(
        jnp.dot(z, wzz_ref[...], preferred_element_type=jnp.float32)
        + jnp.dot(x_ref[...], wzx_ref[...], preferred_element_type=jnp.float32)
        + bz_ref[...])
    ht = jnp.tanh(
        jnp.dot(z, whz_ref[...], preferred_element_type=jnp.float32)
        + jnp.dot(x_ref[...], whx_ref[...], preferred_element_type=jnp.float32)
        + bh_ref[...])
    h_ref[...] = (1.0 - g) * ht

    @pl.when(r == 0)
    def _init():
        zsum_ref[...] = jnp.zeros_like(zsum_ref)

    zsum_ref[...] += jnp.sum(z, axis=0, keepdims=True)

    @pl.when(r == GRID - 1)
    def _head():
        out_ref[...] = (
            jnp.dot(zsum_ref[...] * (1.0 / N), wo_ref[...],
                    preferred_element_type=jnp.float32) + bo_ref[...])


def _tc_fin(acc, xs2, dinv, x, b2, Wzz, Wzx, bz, Whz_, Whx, bh, Wo, bo):
    return pl.pallas_call(
        _fin_body,
        grid=(GRID,),
        in_specs=[
            pl.BlockSpec((2, R, DIN), lambda r: (0, r, 0)),
            pl.BlockSpec((R, DIN), lambda r: (r, 0)),
            pl.BlockSpec((R, DEGW), lambda r: (r, 0)),
            pl.BlockSpec((R, DIN), lambda r: (r, 0)),
            pl.BlockSpec((1, DIN), lambda r: (0, 0)),
            pl.BlockSpec(Wzz.shape, lambda r: (0, 0)),
            pl.BlockSpec(Wzx.shape, lambda r: (0, 0)),
            pl.BlockSpec((1, 256), lambda r: (0, 0)),
            pl.BlockSpec(Whz_.shape, lambda r: (0, 0)),
            pl.BlockSpec(Whx.shape, lambda r: (0, 0)),
            pl.BlockSpec((1, 256), lambda r: (0, 0)),
            pl.BlockSpec(Wo.shape, lambda r: (0, 0)),
            pl.BlockSpec((1, 1), lambda r: (0, 0)),
        ],
        out_specs=[
            pl.BlockSpec((R, 256), lambda r: (r, 0)),
            pl.BlockSpec((1, 1), lambda r: (0, 0)),
            pl.BlockSpec((1, DIN), lambda r: (0, 0)),
        ],
        out_shape=[
            jax.ShapeDtypeStruct((N, 256), jnp.float32),
            jax.ShapeDtypeStruct((1, 1), jnp.float32),
            jax.ShapeDtypeStruct((1, DIN), jnp.float32),
        ],
    )(acc, xs2, dinv, x, b2, Wzz, Wzx, bz, Whz_, Whx, bh, Wo, bo)


def kernel(x, edge_index, W1, b1, W2, b2, Wxz, bxz, Whz, bhz, Wxr, bxr,
           Whr, bhr, Wxh, bxh, Whh, bhh, Wo, bo):
    e = edge_index.shape[1]
    cpt = -(-e // (TILES * C))
    cpt = (cpt + 7) // 8 * 8
    ep = TILES * cpt * C
    src = edge_index[0].astype(jnp.int32)
    dst = edge_index[1].astype(jnp.int32)
    pad_i = jnp.arange(ep - e, dtype=jnp.int32)
    src2d = jnp.concatenate([src, pad_i % 256]).reshape(ep // C, C)
    dst2d = jnp.concatenate([dst, N + pad_i % (NP - N)]).reshape(ep // C, C)

    zfeat = jnp.zeros((NP, DIN), jnp.float32)

    deg = _sc_hist(dst2d, zfeat)
    xs1, dinv = _tc_prep(deg, x)
    acc1 = _sc_prop(xs1, src2d, dst2d, zfeat)
    xs2 = _tc_mid(acc1, xs1[:N], dinv, W1, b1.reshape(1, 256), W2)
    acc2 = _sc_prop(xs2, src2d, dst2d, zfeat)
    H, out, _ = _tc_fin(
        acc2, xs2[:N], dinv, x, b2.reshape(1, DIN),
        Wxz[:DIN], Wxz[DIN:], (bxz + bhz).reshape(1, 256),
        Wxh[:DIN], Wxh[DIN:], (bxh + bhh).reshape(1, 256),
        Wo, bo.reshape(1, 1))
    return (out.reshape(1), H)


# speedup vs baseline: 29.0337x; 1.0235x over previous
import functools

import jax
import jax.numpy as jnp
from jax import lax
from jax.experimental import pallas as pl
from jax.experimental.pallas import tpu as pltpu
from jax.experimental.pallas import tpu_sc as plsc

N = 10000
DIN = 128
C = 128
TILES = 32
RPS = 640
NP = RPS * 16
DEGW = 8
R = 2000
GRID = N // R

_mesh = plsc.VectorSubcoreMesh(core_axis_name="c", subcore_axis_name="s")


def _hist_body(dst2d, zfeat, out, didx, hist2, outrows, idv, shdeg, sem):
    cpt = dst2d.shape[0] // TILES
    nfr = NP // 128
    cid = lax.axis_index("c")
    sid = lax.axis_index("s")
    wid = cid * 16 + sid
    pltpu.sync_copy(zfeat.at[pl.ds(0, nfr)], hist2)

    @pl.when(sid == 0)
    def _zero_shared():
        pltpu.sync_copy(zfeat.at[pl.ds(0, nfr)], shdeg)

    pltpu.sync_copy(dst2d.at[pl.ds(wid * cpt, cpt)], didx)
    iota = lax.iota(jnp.int32, 16)
    for m in range(nfr // 16):
        idv[pl.ds(m * 16, 16)] = iota + m * 16
    ones = jnp.ones((16,), jnp.float32)

    def body(j, carry):
        for k in range(8):
            v = didx[j, pl.ds(k * 16, 16)]
            plsc.addupdate_scatter(hist2, [v >> 7, v & 127], ones)
        return carry

    lax.fori_loop(0, cpt, body, 0)
    plsc.subcore_barrier()
    pltpu.sync_copy(hist2, shdeg.at[idv], add=True)
    plsc.subcore_barrier()

    @pl.when(sid < 10)
    def _expand():
        pltpu.sync_copy(shdeg.at[pl.ds(sid * 8, 8)], hist2.at[pl.ds(0, 8)])
        zero = jnp.zeros((16,), jnp.int32)
        for r in range(8):
            for m in range(8):
                val = hist2[r, pl.ds(m * 16, 16)]
                plsc.store_scatter(outrows, [iota + m * 16, zero], val)
            pltpu.sync_copy(
                outrows, out.at[cid, pl.ds(sid * 1024 + r * 128, 128)])


def _prop_body(xs, src2d, dst2d, zfeat, out, sidx, didx,
               rows0, rows1, shacc, sg0, sg1, ss0, ss1):
    cpt = src2d.shape[0] // TILES
    half = cpt // 2
    npair = half // 2
    cid = lax.axis_index("c")
    sid = lax.axis_index("s")
    wid = cid * 16 + sid
    rbase = sid * RPS
    pltpu.sync_copy(zfeat.at[pl.ds(rbase, RPS)], shacc.at[pl.ds(rbase, RPS)])
    plsc.subcore_barrier()

    for h in range(2):
        pltpu.sync_copy(src2d.at[pl.ds((wid * 2 + h) * half, half)], sidx)
        pltpu.sync_copy(dst2d.at[pl.ds((wid * 2 + h) * half, half)], didx)
        pltpu.async_copy(xs.at[sidx.at[0]], rows0, sg0)

        def pair(k, carry):
            j0 = k * 2
            j1 = j0 + 1
            pltpu.make_async_copy(xs.at[sidx.at[j0]], rows0, sg0).wait()
            pltpu.async_copy(rows0, shacc.at[didx.at[j0]], ss0, add=True)

            @pl.when(k > 0)
            def _():
                pltpu.make_async_copy(
                    rows1, shacc.at[didx.at[j0 - 1]], ss1).wait()

            pltpu.async_copy(xs.at[sidx.at[j1]], rows1, sg1)
            pltpu.make_async_copy(xs.at[sidx.at[j1]], rows1, sg1).wait()
            pltpu.async_copy(rows1, shacc.at[didx.at[j1]], ss1, add=True)
            pltpu.make_async_copy(rows0, shacc.at[didx.at[j0]], ss0).wait()

            @pl.when(k < npair - 1)
            def _():
                pltpu.async_copy(xs.at[sidx.at[j0 + 2]], rows0, sg0)

            return carry

        lax.fori_loop(0, npair, pair, 0)
        pltpu.make_async_copy(rows1, shacc.at[didx.at[half - 1]], ss1).wait()

    plsc.subcore_barrier()
    pltpu.sync_copy(shacc.at[pl.ds(rbase, RPS)], out.at[cid, pl.ds(rbase, RPS)])


def _sc_hist(dst2d, zfeat):
    cpt = dst2d.shape[0] // TILES
    fn = functools.partial(
        pl.kernel,
        mesh=_mesh,
        out_type=jax.ShapeDtypeStruct((2, NP, 8), jnp.float32),
        scratch_types=[
            pltpu.VMEM((cpt, C), jnp.int32),
            pltpu.VMEM((NP // 128, 128), jnp.float32),
            pltpu.VMEM((128, 8), jnp.float32),
            pltpu.VMEM((NP // 128,), jnp.int32),
            pltpu.VMEM_SHARED((NP // 128, 128), jnp.float32),
            pltpu.SemaphoreType.DMA,
        ],
        compiler_params=pltpu.CompilerParams(needs_layout_passes=False),
    )(_hist_body)
    return fn(dst2d, zfeat)


def _sc_prop(xs, src2d, dst2d, zfeat):
    cpt = src2d.shape[0] // TILES
    fn = functools.partial(
        pl.kernel,
        mesh=_mesh,
        out_type=jax.ShapeDtypeStruct((2, NP, DIN), jnp.float32),
        scratch_types=[
            pltpu.VMEM((cpt // 2, C), jnp.int32),
            pltpu.VMEM((cpt // 2, C), jnp.int32),
            pltpu.VMEM((C, DIN), jnp.float32),
            pltpu.VMEM((C, DIN), jnp.float32),
            pltpu.VMEM_SHARED((NP, DIN), jnp.float32),
            pltpu.SemaphoreType.DMA,
            pltpu.SemaphoreType.DMA,
            pltpu.SemaphoreType.DMA,
            pltpu.SemaphoreType.DMA,
        ],
    )(_prop_body)
    return fn(xs, src2d, dst2d, zfeat)


def _prep_body(deg_ref, x_ref, xs_ref, dinv_ref):
    deg = deg_ref[0, :, 0:1] + deg_ref[1, :, 0:1] + 1.0
    dinv = lax.rsqrt(deg)
    dinv_ref[...] = jnp.broadcast_to(dinv, dinv_ref.shape)
    xs_ref[...] = x_ref[...] * dinv


def _tc_prep(deg, x):
    return pl.pallas_call(
        _prep_body,
        grid=(GRID,),
        in_specs=[
            pl.BlockSpec((2, R, 8), lambda r: (0, r, 0)),
            pl.BlockSpec((R, DIN), lambda r: (r, 0)),
        ],
        out_specs=[
            pl.BlockSpec((R, DIN), lambda r: (r, 0)),
            pl.BlockSpec((R, DEGW), lambda r: (r, 0)),
        ],
        out_shape=[
            jax.ShapeDtypeStruct((NP, DIN), jnp.float32),
            jax.ShapeDtypeStruct((N, DEGW), jnp.float32),
        ],
    )(deg, x)


def _mid_body(acc_ref, xs1_ref, dinv_ref, w1_ref, b1_ref, w2_ref, xs2_ref):
    dinv = dinv_ref[:, 0:1]
    s1 = (acc_ref[0] + acc_ref[1] + xs1_ref[...]) * dinv
    f = jnp.maximum(
        jnp.dot(s1, w1_ref[...], preferred_element_type=jnp.float32) + b1_ref[...],
        0.0)
    xw2 = jnp.dot(f, w2_ref[...], preferred_element_type=jnp.float32)
    xs2_ref[...] = xw2 * dinv


def _tc_mid(acc, xs1, dinv, W1, b1, W2):
    return pl.pallas_call(
        _mid_body,
        grid=(GRID,),
        in_specs=[
            pl.BlockSpec((2, R, DIN), lambda r: (0, r, 0)),
            pl.BlockSpec((R, DIN), lambda r: (r, 0)),
            pl.BlockSpec((R, DEGW), lambda r: (r, 0)),
            pl.BlockSpec(W1.shape, lambda r: (0, 0)),
            pl.BlockSpec((1, 256), lambda r: (0, 0)),
            pl.BlockSpec(W2.shape, lambda r: (0, 0)),
        ],
        out_specs=pl.BlockSpec((R, DIN), lambda r: (r, 0)),
        out_shape=jax.ShapeDtypeStruct((NP, DIN), jnp.float32),
    )(acc, xs1, dinv, W1, b1, W2)


def _fin_body(acc_ref, xs2_ref, dinv_ref, x_ref, b2_ref,
              wgz_ref, wgx_ref, bg_ref, wo_ref, bo_ref,
              h_ref, out_ref, zsum_ref):
    r = pl.program_id(0)
    dinv = dinv_ref[:, 0:1]
    z = jnp.maximum(
        (acc_ref[0] + acc_ref[1] + xs2_ref[...]) * dinv + b2_ref[...], 0.0)
    go = (jnp.dot(z, wgz_ref[...], preferred_element_type=jnp.float32)
          + jnp.dot(x_ref[...], wgx_ref[...], preferred_element_type=jnp.float32)
          + bg_ref[...])
    g = jax.nn.sigmoid(go[:, :256])
    ht = jnp.tanh(go[:, 256:])
    h_ref[...] = (1.0 - g) * ht

    @pl.when(r == 0)
    def _init():
        zsum_ref[...] = jnp.zeros_like(zsum_ref)

    zsum_ref[...] += jnp.sum(z, axis=0, keepdims=True)

    @pl.when(r == GRID - 1)
    def _head():
        out_ref[...] = (
            jnp.dot(zsum_ref[...] * (1.0 / N), wo_ref[...],
                    preferred_element_type=jnp.float32) + bo_ref[...])


def _tc_fin(acc, xs2, dinv, x, b2, Wgz, Wgx, bg, Wo, bo):
    return pl.pallas_call(
        _fin_body,
        grid=(GRID,),
        in_specs=[
            pl.BlockSpec((2, R, DIN), lambda r: (0, r, 0)),
            pl.BlockSpec((R, DIN), lambda r: (r, 0)),
            pl.BlockSpec((R, DEGW), lambda r: (r, 0)),
            pl.BlockSpec((R, DIN), lambda r: (r, 0)),
            pl.BlockSpec((1, DIN), lambda r: (0, 0)),
            pl.BlockSpec(Wgz.shape, lambda r: (0, 0)),
            pl.BlockSpec(Wgx.shape, lambda r: (0, 0)),
            pl.BlockSpec((1, 512), lambda r: (0, 0)),
            pl.BlockSpec(Wo.shape, lambda r: (0, 0)),
            pl.BlockSpec((1, 1), lambda r: (0, 0)),
        ],
        out_specs=[
            pl.BlockSpec((R, 256), lambda r: (r, 0)),
            pl.BlockSpec((1, 1), lambda r: (0, 0)),
            pl.BlockSpec((1, DIN), lambda r: (0, 0)),
        ],
        out_shape=[
            jax.ShapeDtypeStruct((N, 256), jnp.float32),
            jax.ShapeDtypeStruct((1, 1), jnp.float32),
            jax.ShapeDtypeStruct((1, DIN), jnp.float32),
        ],
    )(acc, xs2, dinv, x, b2, Wgz, Wgx, bg, Wo, bo)


def kernel(x, edge_index, W1, b1, W2, b2, Wxz, bxz, Whz, bhz, Wxr, bxr,
           Whr, bhr, Wxh, bxh, Whh, bhh, Wo, bo):
    e = edge_index.shape[1]
    cpt = -(-e // (TILES * C))
    cpt = (cpt + 7) // 8 * 8
    ep = TILES * cpt * C
    src = edge_index[0].astype(jnp.int32)
    dst = edge_index[1].astype(jnp.int32)
    pad_i = jnp.arange(ep - e, dtype=jnp.int32)
    src2d = jnp.concatenate([src, pad_i % 256]).reshape(ep // C, C)
    dst2d = jnp.concatenate([dst, N + pad_i % (NP - N)]).reshape(ep // C, C)

    zfeat = jnp.zeros((NP, DIN), jnp.float32)

    deg = _sc_hist(dst2d, zfeat)
    xs1, dinv = _tc_prep(deg, x)
    acc1 = _sc_prop(xs1, src2d, dst2d, zfeat)
    xs2 = _tc_mid(acc1, xs1, dinv, W1, b1.reshape(1, 256), W2)
    acc2 = _sc_prop(xs2, src2d, dst2d, zfeat)
    Wgz = jnp.concatenate([Wxz[:DIN], Wxh[:DIN]], axis=1)
    Wgx = jnp.concatenate([Wxz[DIN:], Wxh[DIN:]], axis=1)
    bg = jnp.concatenate([bxz + bhz, bxh + bhh]).reshape(1, 512)
    H, out, _ = _tc_fin(
        acc2, xs2, dinv, x, b2.reshape(1, DIN), Wgz, Wgx, bg,
        Wo, bo.reshape(1, 1))
    return (out.reshape(1), H)
